# Initial kernel scaffold; baseline (speedup 1.0000x reference)
import jax
import jax.numpy as jnp
from jax.experimental import pallas as pl


def kernel(player_feature_map, player_bbox, ball_feature_map):
    raise NotImplementedError("write your pallas kernel here")


# speedup vs baseline: 1.8751x; 1.8751x over previous
import functools

import jax
import jax.numpy as jnp
from jax.experimental import pallas as pl
from jax.experimental.pallas import tpu as pltpu

_MAX_DET = 100
_BALL_BBOX = 40.0


def _stable_sigmoid(d):
    pos = 1.0 / (1.0 + jnp.exp(-d))
    e = jnp.exp(d)
    neg = e / (1.0 + e)
    return jnp.where(d >= 0, pos, neg)


def _nms_conf(x0, x1, H, W):
    conf = _stable_sigmoid(x1 - x0)
    neg = jnp.float32(-jnp.inf)
    padrow = jnp.full((1, W), neg, jnp.float32)
    up = jnp.concatenate([conf[1:], padrow], axis=0)
    dn = jnp.concatenate([padrow, conf[:-1]], axis=0)
    v = jnp.maximum(conf, jnp.maximum(up, dn))
    padcol = jnp.full((H, 1), neg, jnp.float32)
    lf = jnp.concatenate([v[:, 1:], padcol], axis=1)
    rt = jnp.concatenate([padcol, v[:, :-1]], axis=1)
    pooled = jnp.maximum(v, jnp.maximum(lf, rt))
    return jnp.where(conf == pooled, conf, 0.0)


def _select_loop(cm_ref, H, W, extract_fn):
    cm = cm_ref[...]
    rowmax0 = jnp.max(cm, axis=1, keepdims=True)
    row_iota = jax.lax.broadcasted_iota(jnp.int32, (H, 1), 0)
    lane_iota_w = jax.lax.broadcasted_iota(jnp.int32, (1, W), 1)
    lane128 = jax.lax.broadcasted_iota(jnp.int32, (1, 128), 1)
    zeros128 = jnp.zeros((1, 128), jnp.float32)
    n_extra = len(extract_fn(jnp.int32(0), jnp.int32(0)))

    def body(i, carry):
        rowmax = carry[0]
        accs = carry[1]
        m = jnp.max(rowmax)
        r = jnp.min(jnp.where(rowmax == m, row_iota, H))
        row = cm_ref[pl.ds(r, 1), :]
        c = jnp.min(jnp.where(row == m, lane_iota_w, W))
        extras = extract_fn(r, c)
        newrow = jnp.where(lane_iota_w == c, -1.0, row)
        cm_ref[pl.ds(r, 1), :] = newrow
        rowmax = jnp.where(row_iota == r, jnp.max(newrow), rowmax)
        sel = lane128 == i
        vals = (m, jnp.float32(r), jnp.float32(c)) + extras
        accs = tuple(jnp.where(sel, v, a) for v, a in zip(vals, accs))
        return (rowmax, accs)

    init_accs = tuple(zeros128 for _ in range(3 + n_extra))
    _, accs = jax.lax.fori_loop(0, _MAX_DET, body, (rowmax0, init_accs))
    return accs


def _player_kernel(pfm_ref, pbb_ref, out_ref, cm_ref):
    H, W = 68, 120
    cm_ref[...] = _nms_conf(pfm_ref[0, 0], pfm_ref[0, 1], H, W)
    lane_iota_w = jax.lax.broadcasted_iota(jnp.int32, (1, W), 1)
    scales = (1920.0, 1088.0, 1920.0, 1088.0)

    def extract(r, c):
        out = []
        for ch in range(4):
            brow = pbb_ref[0, ch, pl.ds(r, 1), :]
            out.append(jnp.sum(jnp.where(lane_iota_w == c, brow, 0.0)) * scales[ch])
        return tuple(out)

    val, rf, cf, t0, t1, t2, t3 = _select_loop(cm_ref, H, W, extract)
    ds = 16.0
    xc = cf * ds + (ds - 1.0) / 2.0
    yc = rf * ds + (ds - 1.0) / 2.0
    bx = xc + t0
    by = yc + t1
    zero = jnp.zeros((1, 128), jnp.float32)
    det = jnp.concatenate(
        [bx - 0.5 * t2, by - 0.5 * t3, bx + 0.5 * t2, by + 0.5 * t3, val,
         zero, zero, zero], axis=0)
    out_ref[0] = det


def _ball_kernel(bfm_ref, out_ref, cm_ref):
    H, W = 272, 480
    cm_ref[...] = _nms_conf(bfm_ref[0, 0], bfm_ref[0, 1], H, W)

    def extract(r, c):
        return ()

    val, rf, cf = _select_loop(cm_ref, H, W, extract)
    ds = 4.0
    xc = cf * ds + (ds - 1.0) / 2.0
    yc = rf * ds + (ds - 1.0) / 2.0
    half = 0.5 * _BALL_BBOX
    zero = jnp.zeros((1, 128), jnp.float32)
    det = jnp.concatenate(
        [xc - half, yc - half, xc + half, yc + half, val,
         zero, zero, zero], axis=0)
    out_ref[0] = det


@jax.jit
def kernel(player_feature_map, player_bbox, ball_feature_map):
    B = player_feature_map.shape[0]
    player_out = pl.pallas_call(
        _player_kernel,
        grid=(B,),
        in_specs=[
            pl.BlockSpec((1, 2, 68, 120), lambda b: (b, 0, 0, 0)),
            pl.BlockSpec((1, 4, 68, 120), lambda b: (b, 0, 0, 0)),
        ],
        out_specs=pl.BlockSpec((1, 8, 128), lambda b: (b, 0, 0)),
        out_shape=jax.ShapeDtypeStruct((B, 8, 128), jnp.float32),
        scratch_shapes=[pltpu.VMEM((68, 120), jnp.float32)],
    )(player_feature_map, player_bbox)

    ball_out = pl.pallas_call(
        _ball_kernel,
        grid=(B,),
        in_specs=[pl.BlockSpec((1, 2, 272, 480), lambda b: (b, 0, 0, 0))],
        out_specs=pl.BlockSpec((1, 8, 128), lambda b: (b, 0, 0)),
        out_shape=jax.ShapeDtypeStruct((B, 8, 128), jnp.float32),
        scratch_shapes=[pltpu.VMEM((272, 480), jnp.float32)],
    )(ball_feature_map)

    player_det = jnp.transpose(player_out[:, :5, :_MAX_DET], (0, 2, 1))
    ball_det = jnp.transpose(ball_out[:, :5, :_MAX_DET], (0, 2, 1))
    return jnp.concatenate([player_det, ball_det], axis=1)


# speedup vs baseline: 19.3958x; 10.3441x over previous
import jax
import jax.numpy as jnp
from jax.experimental import pallas as pl
from jax.experimental.pallas import tpu as pltpu

_MAX_DET = 100
_BALL_BBOX = 40.0


def _stable_sigmoid(d):
    pos = 1.0 / (1.0 + jnp.exp(-d))
    e = jnp.exp(d)
    neg = e / (1.0 + e)
    return jnp.where(d >= 0, pos, neg)


def _nms_conf(x0, x1, B, H, W):
    conf = _stable_sigmoid(x1 - x0)
    neg = jnp.float32(-jnp.inf)
    padrow = jnp.full((B, 1, W), neg, jnp.float32)
    up = jnp.concatenate([conf[:, 1:], padrow], axis=1)
    dn = jnp.concatenate([padrow, conf[:, :-1]], axis=1)
    v = jnp.maximum(conf, jnp.maximum(up, dn))
    padcol = jnp.full((B, H, 1), neg, jnp.float32)
    lf = jnp.concatenate([v[:, :, 1:], padcol], axis=2)
    rt = jnp.concatenate([padcol, v[:, :, :-1]], axis=2)
    pooled = jnp.maximum(v, jnp.maximum(lf, rt))
    return jnp.where(conf == pooled, conf, 0.0)


def _select_loop(cm_ref, B, H, W, WAUG, rowmax0, n_extra):
    lane_h = jax.lax.broadcasted_iota(jnp.int32, (1, H), 1)
    lane_w = jax.lax.broadcasted_iota(jnp.int32, (1, WAUG), 1)
    lane128 = jax.lax.broadcasted_iota(jnp.int32, (1, 128), 1)
    zeros = jnp.zeros((B, 128), jnp.float32)

    def body(i, carry):
        rowmax, accs = carry
        mvec = jnp.max(rowmax, axis=1, keepdims=True)
        rvec = jnp.min(jnp.where(rowmax == mvec, lane_h, H),
                       axis=1, keepdims=True)
        rbs = [rvec[b, 0] for b in range(B)]
        rows = jnp.concatenate(
            [cm_ref[b, pl.ds(rbs[b], 1), :] for b in range(B)], axis=0)
        in_map = lane_w < W
        cvec = jnp.min(jnp.where((rows == mvec) & in_map, lane_w, WAUG),
                       axis=1, keepdims=True)
        extras = tuple(
            jnp.sum(jnp.where(lane_w == cvec + 128 * (k + 1), rows, 0.0),
                    axis=1, keepdims=True)
            for k in range(n_extra))
        newrows = jnp.where(lane_w == cvec, -1.0, rows)
        for b in range(B):
            cm_ref[b, pl.ds(rbs[b], 1), :] = newrows[b:b + 1]
        newmax = jnp.max(jnp.where(in_map, newrows, -jnp.inf),
                         axis=1, keepdims=True)
        rowmax = jnp.where(lane_h == rvec, newmax, rowmax)
        sel = lane128 == i
        vals = (mvec, rvec.astype(jnp.float32), cvec.astype(jnp.float32)) + extras
        accs = tuple(jnp.where(sel, v, a) for v, a in zip(vals, accs))
        return (rowmax, accs)

    init = (rowmax0, tuple(zeros for _ in range(3 + n_extra)))
    _, accs = jax.lax.fori_loop(0, _MAX_DET, body, init)
    return accs


def _player_kernel(pfm_ref, pbb_ref, out_ref, aug_ref):
    B, H, W = 16, 68, 120
    WAUG = 640
    cm = _nms_conf(pfm_ref[:, 0], pfm_ref[:, 1], B, H, W)
    rowmax0 = jnp.max(cm, axis=2)
    aug_ref[:, :, 0:W] = cm
    scales = (1920.0, 1088.0, 1920.0, 1088.0)
    for ch in range(4):
        base = 128 * (ch + 1)
        aug_ref[:, :, base:base + W] = pbb_ref[:, ch] * scales[ch]

    val, rf, cf, t0, t1, t2, t3 = _select_loop(
        aug_ref, B, H, W, WAUG, rowmax0, 4)
    ds = 16.0
    xc = cf * ds + (ds - 1.0) / 2.0
    yc = rf * ds + (ds - 1.0) / 2.0
    bx = xc + t0
    by = yc + t1
    zero = jnp.zeros((B, 128), jnp.float32)
    rows = [bx - 0.5 * t2, by - 0.5 * t3, bx + 0.5 * t2, by + 0.5 * t3, val,
            zero, zero, zero]
    out_ref[...] = jnp.concatenate([r[:, None, :] for r in rows], axis=1)


def _ball_kernel(bfm_ref, out_ref, cm_ref):
    B, H, W = 16, 272, 480
    cm = _nms_conf(bfm_ref[:, 0], bfm_ref[:, 1], B, H, W)
    rowmax0 = jnp.max(cm, axis=2)
    cm_ref[...] = cm

    val, rf, cf = _select_loop(cm_ref, B, H, W, W, rowmax0, 0)
    ds = 4.0
    xc = cf * ds + (ds - 1.0) / 2.0
    yc = rf * ds + (ds - 1.0) / 2.0
    half = 0.5 * _BALL_BBOX
    zero = jnp.zeros((B, 128), jnp.float32)
    rows = [xc - half, yc - half, xc + half, yc + half, val,
            zero, zero, zero]
    out_ref[...] = jnp.concatenate([r[:, None, :] for r in rows], axis=1)


@jax.jit
def kernel(player_feature_map, player_bbox, ball_feature_map):
    B = player_feature_map.shape[0]
    player_out = pl.pallas_call(
        _player_kernel,
        out_shape=jax.ShapeDtypeStruct((B, 8, 128), jnp.float32),
        scratch_shapes=[pltpu.VMEM((B, 68, 640), jnp.float32)],
    )(player_feature_map, player_bbox)

    ball_out = pl.pallas_call(
        _ball_kernel,
        out_shape=jax.ShapeDtypeStruct((B, 8, 128), jnp.float32),
        scratch_shapes=[pltpu.VMEM((B, 272, 480), jnp.float32)],
    )(ball_feature_map)

    player_det = jnp.transpose(player_out[:, :5, :_MAX_DET], (0, 2, 1))
    ball_det = jnp.transpose(ball_out[:, :5, :_MAX_DET], (0, 2, 1))
    return jnp.concatenate([player_det, ball_det], axis=1)


# speedup vs baseline: 23.6989x; 1.2219x over previous
import functools

import jax
import jax.numpy as jnp
from jax import lax
from jax.experimental import pallas as pl
from jax.experimental.pallas import tpu as pltpu
from jax.experimental.pallas import tpu_sc as plsc

_MAX_DET = 100
_NCAND = 112
_BALL_BBOX = 40.0


def _stable_sigmoid(d):
    pos = 1.0 / (1.0 + jnp.exp(-d))
    e = jnp.exp(d)
    neg = e / (1.0 + e)
    return jnp.where(d >= 0, pos, neg)


def _nms_conf(x0, x1, B, H, W):
    conf = _stable_sigmoid(x1 - x0)
    neg = jnp.float32(-jnp.inf)
    padrow = jnp.full((B, 1, W), neg, jnp.float32)
    up = jnp.concatenate([conf[:, 1:], padrow], axis=1)
    dn = jnp.concatenate([padrow, conf[:, :-1]], axis=1)
    v = jnp.maximum(conf, jnp.maximum(up, dn))
    padcol = jnp.full((B, H, 1), neg, jnp.float32)
    lf = jnp.concatenate([v[:, :, 1:], padcol], axis=2)
    rt = jnp.concatenate([padcol, v[:, :, :-1]], axis=2)
    pooled = jnp.maximum(v, jnp.maximum(lf, rt))
    return jnp.where(conf == pooled, conf, 0.0)


def _dense_kernel(pfm_ref, bfm_ref, cmp_ref, cmb_ref):
    cmp_ref[...] = _nms_conf(pfm_ref[:, 0], pfm_ref[:, 1], 16, 68, 120)
    cmb_ref[...] = _nms_conf(bfm_ref[:, 0], bfm_ref[:, 1], 16, 272, 480)


def _sc_select(cmp_hbm, cmb_hbm, pbb_hbm,
               pval_o, pidx_o, pbbv_o, bval_o, bidx_o,
               p_data, p_l1, b_data, b_l1, b_l2, pbb_loc,
               pval_b, pidx_b, pbbv_b, bval_b, bidx_b, dma_sem):
    b = lax.axis_index("s")
    h = lax.axis_index("c")
    wid = h * 16 + b
    lane = lax.broadcasted_iota(jnp.int32, (16,), 0)
    zi = jnp.zeros((16,), jnp.int32)
    zf = jnp.zeros((16,), jnp.float32)
    NEG = jnp.float32(-1.0)
    negv = jnp.full((16,), NEG, jnp.float32)

    ball_dma = pltpu.async_copy(
        cmb_hbm.at[pl.ds(b * 130560 + h * 65280, 65280)], b_data, dma_sem)
    pltpu.sync_copy(cmp_hbm.at[pl.ds(b * 8160 + h * 4080, 4080)],
                    p_data.at[pl.ds(0, 4080)])
    p_data[pl.ds(4080, 16)] = negv
    for ch in range(4):
        pltpu.sync_copy(
            pbb_hbm.at[pl.ds(b * 32640 + ch * 8160 + h * 4080, 4080)],
            pbb_loc.at[pl.ds(ch * 4080, 4080)])

    def build_level(src_ref, dst_ref, n_dst_vregs):
        def outer(mi, _):
            rows = (mi * 16 + lane) * 16
            acc = negv
            for c_ in range(16):
                acc = jnp.maximum(acc, plsc.load_gather(src_ref, [rows + c_]))
            dst_ref[pl.ds(mi * 16, 16)] = acc
            return 0
        lax.fori_loop(0, n_dst_vregs, outer, 0)

    def build_top(src_ref):
        acc = negv
        for c_ in range(16):
            acc = jnp.maximum(acc, plsc.load_gather(src_ref, [lane * 16 + c_]))
        return acc

    build_level(p_data, p_l1, 16)
    p_top = build_top(p_l1)

    def ffs_eq(v, m):
        return jnp.min(jnp.where(v == m, lane, 16))

    def select(levels, data_ref, top0, record):
        def step(i, top):
            m = jnp.max(top)
            g = ffs_eq(top, m)
            vregs = []
            idx = g
            for ref in levels:
                v = ref[pl.ds(idx * 16, 16)]
                vregs.append(v)
                idx = idx * 16 + ffs_eq(v, m)
            dv = data_ref[pl.ds(idx * 16, 16)]
            l = ffs_eq(dv, m)
            cell = idx * 16 + l
            record(i, m, cell)
            ndv = jnp.where(lane == l, NEG, dv)
            data_ref[pl.ds(idx * 16, 16)] = ndv
            nm = jnp.max(ndv)
            child = idx
            for ref, v in zip(reversed(levels), reversed(vregs)):
                parent = child // 16
                nv = jnp.where(lane == child - parent * 16, nm, v)
                ref[pl.ds(parent * 16, 16)] = nv
                nm = jnp.max(nv)
                child = parent
            return jnp.where(lane == child, nm, top)
        lax.fori_loop(0, _NCAND, step, top0)

    def rec_player(i, m, cell):
        slot = zi + i
        one = lane == 0
        plsc.store_scatter(pval_b, [slot], zf + m, mask=one)
        plsc.store_scatter(pidx_b, [slot], zi + (h * 4080 + cell), mask=one)
        gidx = jnp.where(lane < 4, cell + lane * 4080, 0)
        bbv = plsc.load_gather(pbb_loc, [gidx])
        plsc.store_scatter(pbbv_b, [i * 4 + lane], bbv, mask=lane < 4)

    def rec_ball(i, m, cell):
        slot = zi + i
        one = lane == 0
        plsc.store_scatter(bval_b, [slot], zf + m, mask=one)
        plsc.store_scatter(bidx_b, [slot], zi + (h * 65280 + cell), mask=one)

    select([p_l1], p_data, p_top, rec_player)

    ball_dma.wait()
    build_level(b_data, b_l1, 255)
    b_l1[pl.ds(4080, 16)] = negv
    build_level(b_l1, b_l2, 16)
    b_top = build_top(b_l2)
    select([b_l2, b_l1], b_data, b_top, rec_ball)

    pltpu.sync_copy(pval_b, pval_o.at[pl.ds(wid * _NCAND, _NCAND)])
    pltpu.sync_copy(pidx_b, pidx_o.at[pl.ds(wid * _NCAND, _NCAND)])
    pltpu.sync_copy(pbbv_b, pbbv_o.at[pl.ds(wid * 4 * _NCAND, 4 * _NCAND)])
    pltpu.sync_copy(bval_b, bval_o.at[pl.ds(wid * _NCAND, _NCAND)])
    pltpu.sync_copy(bidx_b, bidx_o.at[pl.ds(wid * _NCAND, _NCAND)])


def _rank_merge(vA, iA, vB, iB):
    la = lax.broadcasted_iota(jnp.int32, (16, _NCAND), 1)
    vA3 = vA[:, :, None]
    iA3 = iA[:, :, None]
    vB3 = vB[:, None, :]
    iB3 = iB[:, None, :]
    b_over_a = (vB3 > vA3) | ((vB3 == vA3) & (iB3 < iA3))
    rankA = la + jnp.sum(b_over_a.astype(jnp.int32), axis=2)
    a_over_b = (vA3 > vB3) | ((vA3 == vB3) & (iA3 < iB3))
    rankB = la + jnp.sum(a_over_b.astype(jnp.int32), axis=1)
    return rankA, rankB


def _scatter_slots(rankA, rankB, fA, fB):
    slot = lax.broadcasted_iota(jnp.int32, (1, 1, 128), 2)
    mA = rankA[:, :, None] == slot
    mB = rankB[:, :, None] == slot
    zero = jnp.zeros((), fA.dtype)
    return (jnp.sum(jnp.where(mA, fA[:, :, None], zero), axis=1) +
            jnp.sum(jnp.where(mB, fB[:, :, None], zero), axis=1))


def _merge_kernel(pval_ref, pidx_ref, pbbs_ref, bval_ref, bidx_ref,
                  pout_ref, bout_ref):
    zero = jnp.zeros((16, 128), jnp.float32)

    vA, vB = pval_ref[0:16], pval_ref[16:32]
    iA, iB = pidx_ref[0:16], pidx_ref[16:32]
    rankA, rankB = _rank_merge(vA, iA, vB, iB)
    val = _scatter_slots(rankA, rankB, vA, vB)
    idx = _scatter_slots(rankA, rankB, iA, iB)
    ts = []
    scales = (1920.0, 1088.0, 1920.0, 1088.0)
    for ch in range(4):
        t = _scatter_slots(rankA, rankB, pbbs_ref[0:16, ch], pbbs_ref[16:32, ch])
        ts.append(t * scales[ch])
    r = (idx // 120).astype(jnp.float32)
    c = (idx % 120).astype(jnp.float32)
    xc = c * 16.0 + 7.5
    yc = r * 16.0 + 7.5
    bx = xc + ts[0]
    by = yc + ts[1]
    rows = [bx - 0.5 * ts[2], by - 0.5 * ts[3], bx + 0.5 * ts[2],
            by + 0.5 * ts[3], val, zero, zero, zero]
    pout_ref[...] = jnp.concatenate([x[:, None, :] for x in rows], axis=1)

    vA, vB = bval_ref[0:16], bval_ref[16:32]
    iA, iB = bidx_ref[0:16], bidx_ref[16:32]
    rankA, rankB = _rank_merge(vA, iA, vB, iB)
    val = _scatter_slots(rankA, rankB, vA, vB)
    idx = _scatter_slots(rankA, rankB, iA, iB)
    r = (idx // 480).astype(jnp.float32)
    c = (idx % 480).astype(jnp.float32)
    xc = c * 4.0 + 1.5
    yc = r * 4.0 + 1.5
    half = 0.5 * _BALL_BBOX
    rows = [xc - half, yc - half, xc + half, yc + half, val,
            zero, zero, zero]
    bout_ref[...] = jnp.concatenate([x[:, None, :] for x in rows], axis=1)


@jax.jit
def kernel(player_feature_map, player_bbox, ball_feature_map):
    B = player_feature_map.shape[0]
    cmp3, cmb3 = pl.pallas_call(
        _dense_kernel,
        out_shape=[jax.ShapeDtypeStruct((B, 68, 120), jnp.float32),
                   jax.ShapeDtypeStruct((B, 272, 480), jnp.float32)],
    )(player_feature_map, ball_feature_map)

    cmp_flat = cmp3.reshape(B * 8160)
    cmb_flat = cmb3.reshape(B * 130560)
    pbb_flat = player_bbox.reshape(B * 4 * 8160)

    mesh = plsc.VectorSubcoreMesh(core_axis_name="c", subcore_axis_name="s")
    sc = pl.kernel(
        _sc_select,
        out_type=[
            jax.ShapeDtypeStruct((32 * _NCAND,), jnp.float32),
            jax.ShapeDtypeStruct((32 * _NCAND,), jnp.int32),
            jax.ShapeDtypeStruct((32 * 4 * _NCAND,), jnp.float32),
            jax.ShapeDtypeStruct((32 * _NCAND,), jnp.float32),
            jax.ShapeDtypeStruct((32 * _NCAND,), jnp.int32),
        ],
        mesh=mesh,
        compiler_params=pltpu.CompilerParams(needs_layout_passes=False),
        scratch_types=[
            pltpu.VMEM((4096,), jnp.float32),
            pltpu.VMEM((256,), jnp.float32),
            pltpu.VMEM((65280,), jnp.float32),
            pltpu.VMEM((4096,), jnp.float32),
            pltpu.VMEM((256,), jnp.float32),
            pltpu.VMEM((16320,), jnp.float32),
            pltpu.VMEM((_NCAND,), jnp.float32),
            pltpu.VMEM((_NCAND,), jnp.int32),
            pltpu.VMEM((4 * _NCAND,), jnp.float32),
            pltpu.VMEM((_NCAND,), jnp.float32),
            pltpu.VMEM((_NCAND,), jnp.int32),
            pltpu.SemaphoreType.DMA,
        ],
    )
    pval, pidx, pbbv, bval, bidx = sc(cmp_flat, cmb_flat, pbb_flat)
    pval = pval.reshape(32, _NCAND)
    pidx = pidx.reshape(32, _NCAND)
    bval = bval.reshape(32, _NCAND)
    bidx = bidx.reshape(32, _NCAND)
    pbbs = jnp.transpose(pbbv.reshape(32, _NCAND, 4), (0, 2, 1))

    pout, bout = pl.pallas_call(
        _merge_kernel,
        out_shape=[jax.ShapeDtypeStruct((B, 8, 128), jnp.float32),
                   jax.ShapeDtypeStruct((B, 8, 128), jnp.float32)],
    )(pval, pidx, pbbs, bval, bidx)

    player_det = jnp.transpose(pout[:, :5, :_MAX_DET], (0, 2, 1))
    ball_det = jnp.transpose(bout[:, :5, :_MAX_DET], (0, 2, 1))
    return jnp.concatenate([player_det, ball_det], axis=1)


# speedup vs baseline: 23.9837x; 1.0120x over previous
import functools

import jax
import jax.numpy as jnp
from jax import lax
from jax.experimental import pallas as pl
from jax.experimental.pallas import tpu as pltpu
from jax.experimental.pallas import tpu_sc as plsc

_MAX_DET = 100
_NCAND = 104
_BALL_BBOX = 40.0


def _stable_sigmoid(d):
    pos = 1.0 / (1.0 + jnp.exp(-d))
    e = jnp.exp(d)
    neg = e / (1.0 + e)
    return jnp.where(d >= 0, pos, neg)


def _nms_conf(x0, x1, B, H, W):
    conf = _stable_sigmoid(x1 - x0)
    neg = jnp.float32(-jnp.inf)
    padrow = jnp.full((B, 1, W), neg, jnp.float32)
    up = jnp.concatenate([conf[:, 1:], padrow], axis=1)
    dn = jnp.concatenate([padrow, conf[:, :-1]], axis=1)
    v = jnp.maximum(conf, jnp.maximum(up, dn))
    padcol = jnp.full((B, H, 1), neg, jnp.float32)
    lf = jnp.concatenate([v[:, :, 1:], padcol], axis=2)
    rt = jnp.concatenate([padcol, v[:, :, :-1]], axis=2)
    pooled = jnp.maximum(v, jnp.maximum(lf, rt))
    return jnp.where(conf == pooled, conf, 0.0)


def _dense_kernel(pfm_ref, bfm_ref, cmp_ref, cmb_ref):
    cmp_ref[...] = _nms_conf(pfm_ref[:, 0], pfm_ref[:, 1], 1, 68, 120)
    cmb_ref[...] = _nms_conf(bfm_ref[:, 0], bfm_ref[:, 1], 1, 272, 480)


def _sc_select(cmp_hbm, cmb_hbm, pbb_hbm,
               pval_o, pidx_o, pbbv_o, bval_o, bidx_o,
               p_data, p_l1, b_data, b_l1, b_l2, pbb_loc,
               pval_b, pidx_b, pbbv_b, bval_b, bidx_b, dma_sem):
    b = lax.axis_index("s")
    h = lax.axis_index("c")
    wid = h * 16 + b
    lane = lax.broadcasted_iota(jnp.int32, (16,), 0)
    zi = jnp.zeros((16,), jnp.int32)
    zf = jnp.zeros((16,), jnp.float32)
    NEG = jnp.float32(-1.0)
    negv = jnp.full((16,), NEG, jnp.float32)

    ball_dma = pltpu.async_copy(
        cmb_hbm.at[pl.ds(b * 130560 + h * 65280, 65280)], b_data, dma_sem)
    pltpu.sync_copy(cmp_hbm.at[pl.ds(b * 8160 + h * 4080, 4080)],
                    p_data.at[pl.ds(0, 4080)])
    p_data[pl.ds(4080, 16)] = negv
    for ch in range(4):
        pltpu.sync_copy(
            pbb_hbm.at[pl.ds(b * 32640 + ch * 8160 + h * 4080, 4080)],
            pbb_loc.at[pl.ds(ch * 4080, 4080)])

    def build_level(src_ref, dst_ref, n_dst_vregs):
        def outer(mi, _):
            rows = (mi * 16 + lane) * 16
            acc = negv
            for c_ in range(16):
                acc = jnp.maximum(acc, plsc.load_gather(src_ref, [rows + c_]))
            dst_ref[pl.ds(mi * 16, 16)] = acc
            return 0
        lax.fori_loop(0, n_dst_vregs, outer, 0)

    def build_top(src_ref):
        acc = negv
        for c_ in range(16):
            acc = jnp.maximum(acc, plsc.load_gather(src_ref, [lane * 16 + c_]))
        return acc

    build_level(p_data, p_l1, 16)
    p_top = build_top(p_l1)

    def ffs_eq(v, m):
        return jnp.min(jnp.where(v == m, lane, 16))

    def select(levels, data_ref, top0, record):
        def step(i, top):
            m = jnp.max(top)
            g = ffs_eq(top, m)
            vregs = []
            idx = g
            for ref in levels:
                v = ref[pl.ds(idx * 16, 16)]
                vregs.append(v)
                idx = idx * 16 + ffs_eq(v, m)
            dv = data_ref[pl.ds(idx * 16, 16)]
            l = ffs_eq(dv, m)
            cell = idx * 16 + l
            record(i, m, cell)
            ndv = jnp.where(lane == l, NEG, dv)
            data_ref[pl.ds(idx * 16, 16)] = ndv
            nm = jnp.max(ndv)
            child = idx
            for ref, v in zip(reversed(levels), reversed(vregs)):
                parent = child // 16
                nv = jnp.where(lane == child - parent * 16, nm, v)
                ref[pl.ds(parent * 16, 16)] = nv
                nm = jnp.max(nv)
                child = parent
            return jnp.where(lane == child, nm, top)
        lax.fori_loop(0, _NCAND, step, top0)

    def rec_player(i, m, cell):
        slot = zi + i
        one = lane == 0
        plsc.store_scatter(pval_b, [slot], zf + m, mask=one)
        plsc.store_scatter(pidx_b, [slot], zi + (h * 4080 + cell), mask=one)
        gidx = jnp.where(lane < 4, cell + lane * 4080, 0)
        bbv = plsc.load_gather(pbb_loc, [gidx])
        plsc.store_scatter(pbbv_b, [i * 4 + lane], bbv, mask=lane < 4)

    def rec_ball(i, m, cell):
        slot = zi + i
        one = lane == 0
        plsc.store_scatter(bval_b, [slot], zf + m, mask=one)
        plsc.store_scatter(bidx_b, [slot], zi + (h * 65280 + cell), mask=one)

    select([p_l1], p_data, p_top, rec_player)

    ball_dma.wait()
    build_level(b_data, b_l1, 255)
    b_l1[pl.ds(4080, 16)] = negv
    build_level(b_l1, b_l2, 16)
    b_top = build_top(b_l2)
    select([b_l2, b_l1], b_data, b_top, rec_ball)

    pltpu.sync_copy(pval_b, pval_o.at[pl.ds(wid * _NCAND, _NCAND)])
    pltpu.sync_copy(pidx_b, pidx_o.at[pl.ds(wid * _NCAND, _NCAND)])
    pltpu.sync_copy(pbbv_b, pbbv_o.at[pl.ds(wid * 4 * _NCAND, 4 * _NCAND)])
    pltpu.sync_copy(bval_b, bval_o.at[pl.ds(wid * _NCAND, _NCAND)])
    pltpu.sync_copy(bidx_b, bidx_o.at[pl.ds(wid * _NCAND, _NCAND)])


def _rank_merge(vA, iA, vB, iB):
    la = lax.broadcasted_iota(jnp.int32, (16, _NCAND), 1)
    vA3 = vA[:, :, None]
    iA3 = iA[:, :, None]
    vB3 = vB[:, None, :]
    iB3 = iB[:, None, :]
    b_over_a = (vB3 > vA3) | ((vB3 == vA3) & (iB3 < iA3))
    rankA = la + jnp.sum(b_over_a.astype(jnp.int32), axis=2)
    a_over_b = (vA3 > vB3) | ((vA3 == vB3) & (iA3 < iB3))
    rankB = la + jnp.sum(a_over_b.astype(jnp.int32), axis=1)
    return rankA, rankB


def _scatter_slots(rankA, rankB, fA, fB):
    slot = lax.broadcasted_iota(jnp.int32, (1, 1, 128), 2)
    mA = rankA[:, :, None] == slot
    mB = rankB[:, :, None] == slot
    zero = jnp.zeros((), fA.dtype)
    return (jnp.sum(jnp.where(mA, fA[:, :, None], zero), axis=1) +
            jnp.sum(jnp.where(mB, fB[:, :, None], zero), axis=1))


def _merge_kernel(pval_ref, pidx_ref, pbbs_ref, bval_ref, bidx_ref,
                  pout_ref, bout_ref):
    zero = jnp.zeros((16, 128), jnp.float32)

    vA, vB = pval_ref[0:16], pval_ref[16:32]
    iA, iB = pidx_ref[0:16], pidx_ref[16:32]
    rankA, rankB = _rank_merge(vA, iA, vB, iB)
    val = _scatter_slots(rankA, rankB, vA, vB)
    idx = _scatter_slots(rankA, rankB, iA, iB)
    ts = []
    scales = (1920.0, 1088.0, 1920.0, 1088.0)
    for ch in range(4):
        t = _scatter_slots(rankA, rankB, pbbs_ref[0:16, ch], pbbs_ref[16:32, ch])
        ts.append(t * scales[ch])
    r = (idx // 120).astype(jnp.float32)
    c = (idx % 120).astype(jnp.float32)
    xc = c * 16.0 + 7.5
    yc = r * 16.0 + 7.5
    bx = xc + ts[0]
    by = yc + ts[1]
    rows = [bx - 0.5 * ts[2], by - 0.5 * ts[3], bx + 0.5 * ts[2],
            by + 0.5 * ts[3], val, zero, zero, zero]
    pout_ref[...] = jnp.concatenate([x[:, None, :] for x in rows], axis=1)

    vA, vB = bval_ref[0:16], bval_ref[16:32]
    iA, iB = bidx_ref[0:16], bidx_ref[16:32]
    rankA, rankB = _rank_merge(vA, iA, vB, iB)
    val = _scatter_slots(rankA, rankB, vA, vB)
    idx = _scatter_slots(rankA, rankB, iA, iB)
    r = (idx // 480).astype(jnp.float32)
    c = (idx % 480).astype(jnp.float32)
    xc = c * 4.0 + 1.5
    yc = r * 4.0 + 1.5
    half = 0.5 * _BALL_BBOX
    rows = [xc - half, yc - half, xc + half, yc + half, val,
            zero, zero, zero]
    bout_ref[...] = jnp.concatenate([x[:, None, :] for x in rows], axis=1)


@jax.jit
def kernel(player_feature_map, player_bbox, ball_feature_map):
    B = player_feature_map.shape[0]
    cmp3, cmb3 = pl.pallas_call(
        _dense_kernel,
        grid=(B,),
        in_specs=[pl.BlockSpec((1, 2, 68, 120), lambda i: (i, 0, 0, 0)),
                  pl.BlockSpec((1, 2, 272, 480), lambda i: (i, 0, 0, 0))],
        out_specs=[pl.BlockSpec((1, 68, 120), lambda i: (i, 0, 0)),
                   pl.BlockSpec((1, 272, 480), lambda i: (i, 0, 0))],
        out_shape=[jax.ShapeDtypeStruct((B, 68, 120), jnp.float32),
                   jax.ShapeDtypeStruct((B, 272, 480), jnp.float32)],
    )(player_feature_map, ball_feature_map)

    cmp_flat = cmp3.reshape(B * 8160)
    cmb_flat = cmb3.reshape(B * 130560)
    pbb_flat = player_bbox.reshape(B * 4 * 8160)

    mesh = plsc.VectorSubcoreMesh(core_axis_name="c", subcore_axis_name="s")
    sc = pl.kernel(
        _sc_select,
        out_type=[
            jax.ShapeDtypeStruct((32 * _NCAND,), jnp.float32),
            jax.ShapeDtypeStruct((32 * _NCAND,), jnp.int32),
            jax.ShapeDtypeStruct((32 * 4 * _NCAND,), jnp.float32),
            jax.ShapeDtypeStruct((32 * _NCAND,), jnp.float32),
            jax.ShapeDtypeStruct((32 * _NCAND,), jnp.int32),
        ],
        mesh=mesh,
        compiler_params=pltpu.CompilerParams(needs_layout_passes=False),
        scratch_types=[
            pltpu.VMEM((4096,), jnp.float32),
            pltpu.VMEM((256,), jnp.float32),
            pltpu.VMEM((65280,), jnp.float32),
            pltpu.VMEM((4096,), jnp.float32),
            pltpu.VMEM((256,), jnp.float32),
            pltpu.VMEM((16320,), jnp.float32),
            pltpu.VMEM((_NCAND,), jnp.float32),
            pltpu.VMEM((_NCAND,), jnp.int32),
            pltpu.VMEM((4 * _NCAND,), jnp.float32),
            pltpu.VMEM((_NCAND,), jnp.float32),
            pltpu.VMEM((_NCAND,), jnp.int32),
            pltpu.SemaphoreType.DMA,
        ],
    )
    pval, pidx, pbbv, bval, bidx = sc(cmp_flat, cmb_flat, pbb_flat)
    pval = pval.reshape(32, _NCAND)
    pidx = pidx.reshape(32, _NCAND)
    bval = bval.reshape(32, _NCAND)
    bidx = bidx.reshape(32, _NCAND)
    pbbs = jnp.transpose(pbbv.reshape(32, _NCAND, 4), (0, 2, 1))

    pout, bout = pl.pallas_call(
        _merge_kernel,
        out_shape=[jax.ShapeDtypeStruct((B, 8, 128), jnp.float32),
                   jax.ShapeDtypeStruct((B, 8, 128), jnp.float32)],
    )(pval, pidx, pbbs, bval, bidx)

    player_det = jnp.transpose(pout[:, :5, :_MAX_DET], (0, 2, 1))
    ball_det = jnp.transpose(bout[:, :5, :_MAX_DET], (0, 2, 1))
    return jnp.concatenate([player_det, ball_det], axis=1)


# speedup vs baseline: 25.1654x; 1.0493x over previous
import functools

import jax
import jax.numpy as jnp
from jax import lax
from jax.experimental import pallas as pl
from jax.experimental.pallas import tpu as pltpu
from jax.experimental.pallas import tpu_sc as plsc

_MAX_DET = 100
_NCAND = 104
_BALL_BBOX = 40.0


def _stable_sigmoid(d):
    pos = 1.0 / (1.0 + jnp.exp(-d))
    e = jnp.exp(d)
    neg = e / (1.0 + e)
    return jnp.where(d >= 0, pos, neg)


def _nms_conf(x0, x1, B, H, W):
    conf = _stable_sigmoid(x1 - x0)
    neg = jnp.float32(-jnp.inf)
    padrow = jnp.full((B, 1, W), neg, jnp.float32)
    up = jnp.concatenate([conf[:, 1:], padrow], axis=1)
    dn = jnp.concatenate([padrow, conf[:, :-1]], axis=1)
    v = jnp.maximum(conf, jnp.maximum(up, dn))
    padcol = jnp.full((B, H, 1), neg, jnp.float32)
    lf = jnp.concatenate([v[:, :, 1:], padcol], axis=2)
    rt = jnp.concatenate([padcol, v[:, :, :-1]], axis=2)
    pooled = jnp.maximum(v, jnp.maximum(lf, rt))
    return jnp.where(conf == pooled, conf, 0.0)


def _dense_kernel(pfm_ref, bfm_ref, cmp_ref, cmb_ref):
    cmp_ref[...] = _nms_conf(pfm_ref[:, 0], pfm_ref[:, 1], 1, 68, 120)
    cmb_ref[...] = _nms_conf(bfm_ref[:, 0], bfm_ref[:, 1], 1, 272, 480)


def _sc_select(cmp_hbm, cmb_hbm, pbb_hbm,
               pval_o, pidx_o, pbbv_o, bval_o, bidx_o,
               p_data, p_l1, b_data, b_l1, b_l2, pbb_loc,
               pval_b, pidx_b, pbbv_b, bval_b, bidx_b, dma_sem):
    b = lax.axis_index("s")
    h = lax.axis_index("c")
    wid = h * 16 + b
    lane = lax.broadcasted_iota(jnp.int32, (16,), 0)
    zi = jnp.zeros((16,), jnp.int32)
    zf = jnp.zeros((16,), jnp.float32)
    NEG = jnp.float32(-1.0)
    negv = jnp.full((16,), NEG, jnp.float32)

    ball_dma = pltpu.async_copy(
        cmb_hbm.at[pl.ds(b * 130560 + h * 65280, 65280)], b_data, dma_sem)
    pltpu.sync_copy(cmp_hbm.at[pl.ds(b * 8160 + h * 4080, 4080)],
                    p_data.at[pl.ds(0, 4080)])
    p_data[pl.ds(4080, 16)] = negv
    for ch in range(4):
        pltpu.sync_copy(
            pbb_hbm.at[pl.ds(b * 32640 + ch * 8160 + h * 4080, 4080)],
            pbb_loc.at[pl.ds(ch * 4080, 4080)])

    def build_level(src_ref, dst_ref, n_dst_vregs):
        def outer(mi, _):
            rows = (mi * 16 + lane) * 16
            acc = negv
            for c_ in range(16):
                acc = jnp.maximum(acc, plsc.load_gather(src_ref, [rows + c_]))
            dst_ref[pl.ds(mi * 16, 16)] = acc
            return 0
        lax.fori_loop(0, n_dst_vregs, outer, 0)

    def build_top(src_ref):
        acc = negv
        for c_ in range(16):
            acc = jnp.maximum(acc, plsc.load_gather(src_ref, [lane * 16 + c_]))
        return acc

    build_level(p_data, p_l1, 16)
    p_top = build_top(p_l1)

    def ffs_eq(v, m):
        return jnp.min(jnp.where(v == m, lane, 16))

    def make_step(levels, data_ref, record):
        def step(i, top):
            m = jnp.max(top)
            g = ffs_eq(top, m)
            vregs = []
            idx = g
            for ref in levels:
                v = ref[pl.ds(idx * 16, 16)]
                vregs.append(v)
                idx = idx * 16 + ffs_eq(v, m)
            dv = data_ref[pl.ds(idx * 16, 16)]
            l = ffs_eq(dv, m)
            cell = idx * 16 + l
            record(i, m, cell)
            ndv = jnp.where(lane == l, NEG, dv)
            data_ref[pl.ds(idx * 16, 16)] = ndv
            nm = jnp.max(ndv)
            child = idx
            for ref, v in zip(reversed(levels), reversed(vregs)):
                parent = child // 16
                nv = jnp.where(lane == child - parent * 16, nm, v)
                ref[pl.ds(parent * 16, 16)] = nv
                nm = jnp.max(nv)
                child = parent
            return jnp.where(lane == child, nm, top)
        return step

    def rec_player(i, m, cell):
        slot = zi + i
        one = lane == 0
        plsc.store_scatter(pval_b, [slot], zf + m, mask=one)
        plsc.store_scatter(pidx_b, [slot], zi + (h * 4080 + cell), mask=one)
        gidx = jnp.where(lane < 4, cell + lane * 4080, 0)
        bbv = plsc.load_gather(pbb_loc, [gidx])
        plsc.store_scatter(pbbv_b, [i * 4 + lane], bbv, mask=lane < 4)

    def rec_ball(i, m, cell):
        slot = zi + i
        one = lane == 0
        plsc.store_scatter(bval_b, [slot], zf + m, mask=one)
        plsc.store_scatter(bidx_b, [slot], zi + (h * 65280 + cell), mask=one)

    ball_dma.wait()
    build_level(b_data, b_l1, 255)
    b_l1[pl.ds(4080, 16)] = negv
    build_level(b_l1, b_l2, 16)
    b_top = build_top(b_l2)

    step_p = make_step([p_l1], p_data, rec_player)
    step_b = make_step([b_l2, b_l1], b_data, rec_ball)

    def step_both(i, carry):
        tp, tb = carry
        return (step_p(i, tp), step_b(i, tb))

    lax.fori_loop(0, _NCAND, step_both, (p_top, b_top))

    pltpu.sync_copy(pval_b, pval_o.at[pl.ds(wid * _NCAND, _NCAND)])
    pltpu.sync_copy(pidx_b, pidx_o.at[pl.ds(wid * _NCAND, _NCAND)])
    pltpu.sync_copy(pbbv_b, pbbv_o.at[pl.ds(wid * 4 * _NCAND, 4 * _NCAND)])
    pltpu.sync_copy(bval_b, bval_o.at[pl.ds(wid * _NCAND, _NCAND)])
    pltpu.sync_copy(bidx_b, bidx_o.at[pl.ds(wid * _NCAND, _NCAND)])


def _rank_merge(vA, iA, vB, iB):
    la = lax.broadcasted_iota(jnp.int32, (16, _NCAND), 1)
    vA3 = vA[:, :, None]
    iA3 = iA[:, :, None]
    vB3 = vB[:, None, :]
    iB3 = iB[:, None, :]
    b_over_a = (vB3 > vA3) | ((vB3 == vA3) & (iB3 < iA3))
    rankA = la + jnp.sum(b_over_a.astype(jnp.int32), axis=2)
    a_over_b = (vA3 > vB3) | ((vA3 == vB3) & (iA3 < iB3))
    rankB = la + jnp.sum(a_over_b.astype(jnp.int32), axis=1)
    return rankA, rankB


def _scatter_slots(rankA, rankB, fA, fB):
    slot = lax.broadcasted_iota(jnp.int32, (1, 1, 128), 2)
    mA = rankA[:, :, None] == slot
    mB = rankB[:, :, None] == slot
    zero = jnp.zeros((), fA.dtype)
    return (jnp.sum(jnp.where(mA, fA[:, :, None], zero), axis=1) +
            jnp.sum(jnp.where(mB, fB[:, :, None], zero), axis=1))


def _merge_kernel(pval_ref, pidx_ref, pbbs_ref, bval_ref, bidx_ref,
                  pout_ref, bout_ref):
    zero = jnp.zeros((16, 128), jnp.float32)

    vA, vB = pval_ref[0:16], pval_ref[16:32]
    iA, iB = pidx_ref[0:16], pidx_ref[16:32]
    rankA, rankB = _rank_merge(vA, iA, vB, iB)
    val = _scatter_slots(rankA, rankB, vA, vB)
    idx = _scatter_slots(rankA, rankB, iA, iB)
    ts = []
    scales = (1920.0, 1088.0, 1920.0, 1088.0)
    for ch in range(4):
        t = _scatter_slots(rankA, rankB, pbbs_ref[0:16, ch], pbbs_ref[16:32, ch])
        ts.append(t * scales[ch])
    r = (idx // 120).astype(jnp.float32)
    c = (idx % 120).astype(jnp.float32)
    xc = c * 16.0 + 7.5
    yc = r * 16.0 + 7.5
    bx = xc + ts[0]
    by = yc + ts[1]
    rows = [bx - 0.5 * ts[2], by - 0.5 * ts[3], bx + 0.5 * ts[2],
            by + 0.5 * ts[3], val, zero, zero, zero]
    pout_ref[...] = jnp.concatenate([x[:, None, :] for x in rows], axis=1)

    vA, vB = bval_ref[0:16], bval_ref[16:32]
    iA, iB = bidx_ref[0:16], bidx_ref[16:32]
    rankA, rankB = _rank_merge(vA, iA, vB, iB)
    val = _scatter_slots(rankA, rankB, vA, vB)
    idx = _scatter_slots(rankA, rankB, iA, iB)
    r = (idx // 480).astype(jnp.float32)
    c = (idx % 480).astype(jnp.float32)
    xc = c * 4.0 + 1.5
    yc = r * 4.0 + 1.5
    half = 0.5 * _BALL_BBOX
    rows = [xc - half, yc - half, xc + half, yc + half, val,
            zero, zero, zero]
    bout_ref[...] = jnp.concatenate([x[:, None, :] for x in rows], axis=1)


@jax.jit
def kernel(player_feature_map, player_bbox, ball_feature_map):
    B = player_feature_map.shape[0]
    cmp3, cmb3 = pl.pallas_call(
        _dense_kernel,
        grid=(B,),
        in_specs=[pl.BlockSpec((1, 2, 68, 120), lambda i: (i, 0, 0, 0)),
                  pl.BlockSpec((1, 2, 272, 480), lambda i: (i, 0, 0, 0))],
        out_specs=[pl.BlockSpec((1, 68, 120), lambda i: (i, 0, 0)),
                   pl.BlockSpec((1, 272, 480), lambda i: (i, 0, 0))],
        out_shape=[jax.ShapeDtypeStruct((B, 68, 120), jnp.float32),
                   jax.ShapeDtypeStruct((B, 272, 480), jnp.float32)],
    )(player_feature_map, ball_feature_map)

    cmp_flat = cmp3.reshape(B * 8160)
    cmb_flat = cmb3.reshape(B * 130560)
    pbb_flat = player_bbox.reshape(B * 4 * 8160)

    mesh = plsc.VectorSubcoreMesh(core_axis_name="c", subcore_axis_name="s")
    sc = pl.kernel(
        _sc_select,
        out_type=[
            jax.ShapeDtypeStruct((32 * _NCAND,), jnp.float32),
            jax.ShapeDtypeStruct((32 * _NCAND,), jnp.int32),
            jax.ShapeDtypeStruct((32 * 4 * _NCAND,), jnp.float32),
            jax.ShapeDtypeStruct((32 * _NCAND,), jnp.float32),
            jax.ShapeDtypeStruct((32 * _NCAND,), jnp.int32),
        ],
        mesh=mesh,
        compiler_params=pltpu.CompilerParams(needs_layout_passes=False),
        scratch_types=[
            pltpu.VMEM((4096,), jnp.float32),
            pltpu.VMEM((256,), jnp.float32),
            pltpu.VMEM((65280,), jnp.float32),
            pltpu.VMEM((4096,), jnp.float32),
            pltpu.VMEM((256,), jnp.float32),
            pltpu.VMEM((16320,), jnp.float32),
            pltpu.VMEM((_NCAND,), jnp.float32),
            pltpu.VMEM((_NCAND,), jnp.int32),
            pltpu.VMEM((4 * _NCAND,), jnp.float32),
            pltpu.VMEM((_NCAND,), jnp.float32),
            pltpu.VMEM((_NCAND,), jnp.int32),
            pltpu.SemaphoreType.DMA,
        ],
    )
    pval, pidx, pbbv, bval, bidx = sc(cmp_flat, cmb_flat, pbb_flat)
    pval = pval.reshape(32, _NCAND)
    pidx = pidx.reshape(32, _NCAND)
    bval = bval.reshape(32, _NCAND)
    bidx = bidx.reshape(32, _NCAND)
    pbbs = jnp.transpose(pbbv.reshape(32, _NCAND, 4), (0, 2, 1))

    pout, bout = pl.pallas_call(
        _merge_kernel,
        out_shape=[jax.ShapeDtypeStruct((B, 8, 128), jnp.float32),
                   jax.ShapeDtypeStruct((B, 8, 128), jnp.float32)],
    )(pval, pidx, pbbs, bval, bidx)

    player_det = jnp.transpose(pout[:, :5, :_MAX_DET], (0, 2, 1))
    ball_det = jnp.transpose(bout[:, :5, :_MAX_DET], (0, 2, 1))
    return jnp.concatenate([player_det, ball_det], axis=1)
